# Initial kernel scaffold; baseline (speedup 1.0000x reference)
import jax
import jax.numpy as jnp
from jax.experimental import pallas as pl


def kernel(x, edge_index, W1, b1, W2, b2):
    raise NotImplementedError("write your pallas kernel here")


# speedup vs baseline: 10.6599x; 10.6599x over previous
import functools

import jax
import jax.numpy as jnp
from jax import lax
from jax.experimental import pallas as pl
from jax.experimental.pallas import tpu as pltpu
from jax.experimental.pallas import tpu_sc as plsc

N = 10000
D = 128
E = 320000
NW = 32
NPAD = 10240
EC = 128
ER = 2528
EPAD = EC * ER
RPT = ER // NW
TROWS = NPAD // NW

_MESH = plsc.VectorSubcoreMesh(core_axis_name="c", subcore_axis_name="s")


@functools.partial(
    pl.kernel,
    out_type=jax.ShapeDtypeStruct((2 * NPAD,), jnp.float32),
    mesh=_MESH,
    scratch_types=[
        pltpu.VMEM_SHARED((NPAD,), jnp.float32),
        pltpu.VMEM((TROWS,), jnp.float32),
        pltpu.VMEM((EC,), jnp.float32),
        pltpu.VMEM((EC,), jnp.int32),
    ],
)
def _sc_degree(dst_hbm, out_hbm, acc, buf, ones, dstv):
    c = lax.axis_index("c")
    s = lax.axis_index("s")
    wid = c * 16 + s
    zv = jnp.zeros((16,), jnp.float32)
    ov = jnp.ones((16,), jnp.float32)
    for q in range(TROWS // 16):
        buf[pl.ds(q * 16, 16)] = zv
    for q in range(EC // 16):
        ones[pl.ds(q * 16, 16)] = ov
    pltpu.sync_copy(buf, acc.at[pl.ds(s * TROWS, TROWS)])
    plsc.subcore_barrier()

    def body(i, carry):
        pltpu.sync_copy(dst_hbm.at[wid * RPT + i], dstv)
        pltpu.sync_copy(ones, acc.at[dstv], add=True)
        return carry

    lax.fori_loop(0, RPT, body, 0)
    plsc.subcore_barrier()
    pltpu.sync_copy(acc.at[pl.ds(s * TROWS, TROWS)], buf)
    pltpu.sync_copy(buf, out_hbm.at[pl.ds(c * NPAD + s * TROWS, TROWS)])


@functools.partial(
    pl.kernel,
    out_type=jax.ShapeDtypeStruct((2 * NPAD, D), jnp.float32),
    mesh=_MESH,
    scratch_types=[
        pltpu.VMEM_SHARED((NPAD, D), jnp.float32),
        pltpu.VMEM((16, D), jnp.float32),
        pltpu.VMEM((EC,), jnp.int32),
        pltpu.VMEM((EC,), jnp.int32),
        pltpu.VMEM((EC, D), jnp.float32),
        pltpu.SemaphoreType.DMA,
    ],
)
def _sc_agg(src_hbm, dst_hbm, tab_hbm, out_hbm, acc, zbuf, srcv, dstv, rows,
            sem):
    c = lax.axis_index("c")
    s = lax.axis_index("s")
    wid = c * 16 + s
    zv = jnp.zeros((16,), jnp.float32)
    for r in range(16):
        for q in range(D // 16):
            zbuf[r, pl.ds(q * 16, 16)] = zv

    def zloop(i, carry):
        pltpu.sync_copy(zbuf, acc.at[pl.ds(s * TROWS + i * 16, 16)])
        return carry

    lax.fori_loop(0, TROWS // 16, zloop, 0)
    plsc.subcore_barrier()

    def body(i, carry):
        r = wid * RPT + i
        pltpu.sync_copy(src_hbm.at[r], srcv)
        pltpu.sync_copy(dst_hbm.at[r], dstv)
        pltpu.async_copy(tab_hbm.at[srcv], rows, sem).wait()
        pltpu.sync_copy(rows, acc.at[dstv], add=True)
        return carry

    lax.fori_loop(0, RPT, body, 0)
    plsc.subcore_barrier()
    pltpu.sync_copy(acc.at[pl.ds(s * TROWS, TROWS)],
                    out_hbm.at[pl.ds(c * NPAD + s * TROWS, TROWS)])


_R = 1000


def _tc1_body(d0_ref, d1_ref, x_ref, w_ref, hs_ref, dis_ref):
    dis = lax.rsqrt(d0_ref[...] + d1_ref[...] + 1.0)
    h = jnp.dot(x_ref[...], w_ref[...], preferred_element_type=jnp.float32)
    hs_ref[...] = h * dis
    dis_ref[...] = dis


def _tc2_body(p0_ref, p1_ref, hs_ref, dis_ref, b_ref, w_ref, out_ref):
    dis = dis_ref[...]
    o1 = jnp.maximum(
        dis * (p0_ref[...] + p1_ref[...] + hs_ref[...]) + b_ref[...], 0.0)
    out_ref[...] = jnp.dot(
        o1, w_ref[...], preferred_element_type=jnp.float32) * dis


def _tc3_body(q0_ref, q1_ref, hs_ref, dis_ref, b_ref, out_ref):
    out_ref[...] = dis_ref[...] * (
        q0_ref[...] + q1_ref[...] + hs_ref[...]) + b_ref[...]


def _row_spec(width):
    return pl.BlockSpec((_R, width), lambda i: (i, 0))


def _full_spec(rows, cols):
    return pl.BlockSpec((rows, cols), lambda i: (0, 0))


def _tc1(d0, d1, x, w):
    return pl.pallas_call(
        _tc1_body,
        grid=(N // _R,),
        in_specs=[_row_spec(1), _row_spec(1), _row_spec(D), _full_spec(D, D)],
        out_specs=[_row_spec(D), _row_spec(1)],
        out_shape=[
            jax.ShapeDtypeStruct((N, D), jnp.float32),
            jax.ShapeDtypeStruct((N, 1), jnp.float32),
        ],
    )(d0, d1, x, w)


def _tc2(p0, p1, hs, dis, b, w):
    return pl.pallas_call(
        _tc2_body,
        grid=(N // _R,),
        in_specs=[_row_spec(D), _row_spec(D), _row_spec(D), _row_spec(1),
                  _full_spec(1, D), _full_spec(D, D)],
        out_specs=_row_spec(D),
        out_shape=jax.ShapeDtypeStruct((N, D), jnp.float32),
    )(p0, p1, hs, dis, b, w)


def _tc3(q0, q1, hs, dis, b):
    return pl.pallas_call(
        _tc3_body,
        grid=(N // _R,),
        in_specs=[_row_spec(D), _row_spec(D), _row_spec(D), _row_spec(1),
                  _full_spec(1, D)],
        out_specs=_row_spec(D),
        out_shape=jax.ShapeDtypeStruct((N, D), jnp.float32),
    )(q0, q1, hs, dis, b)


def kernel(x, edge_index, W1, b1, W2, b2):
    src = edge_index[0].astype(jnp.int32)
    dst = edge_index[1].astype(jnp.int32)
    pad = EPAD - E
    src2d = jnp.concatenate([src, jnp.zeros((pad,), jnp.int32)]).reshape(ER, EC)
    dst2d = jnp.concatenate([dst, jnp.full((pad,), N, jnp.int32)]).reshape(ER, EC)

    degp = _sc_degree(dst2d).reshape(2, NPAD)
    d0 = degp[0, :N].reshape(N, 1)
    d1 = degp[1, :N].reshape(N, 1)

    hs1, dis = _tc1(d0, d1, x, W1)
    p = _sc_agg(src2d, dst2d, hs1).reshape(2, NPAD, D)
    hs2 = _tc2(p[0, :N], p[1, :N], hs1, dis, b1.reshape(1, D), W2)
    q = _sc_agg(src2d, dst2d, hs2).reshape(2, NPAD, D)
    return _tc3(q[0, :N], q[1, :N], hs2, dis, b2.reshape(1, D))
